# baseline (speedup 1.0000x reference)
import functools

import jax
import jax.numpy as jnp
from jax import lax
from jax.experimental import pallas as pl
from jax.experimental.pallas import tpu as pltpu
from jax.experimental.pallas import tpu_sc as plsc

NS = 16
B = 112
CH = 15


def _sc_agg_build(n, epad):
    dh = 128
    nrow_pad = n + 8
    nchunk = epad // (NS * CH * B)
    rps = n // NS

    mesh = plsc.VectorSubcoreMesh(core_axis_name="c", subcore_axis_name="s",
                                  num_cores=2, num_subcores=NS)

    @functools.partial(
        pl.kernel,
        out_type=[
            jax.ShapeDtypeStruct((n, dh), jnp.float32),
            jax.ShapeDtypeStruct((n, dh), jnp.float32),
        ],
        mesh=mesh,
        scratch_types=[
            pltpu.VMEM_SHARED((nrow_pad, dh), jnp.float32),
            pltpu.VMEM((2, CH, B), jnp.int32),
            pltpu.VMEM((2, CH, B), jnp.int32),
            pltpu.VMEM((3, B, dh), jnp.float32),
            pltpu.SemaphoreType.DMA,
            pltpu.SemaphoreType.DMA,
            pltpu.SemaphoreType.DMA,
            pltpu.SemaphoreType.DMA,
            pltpu.SemaphoreType.DMA,
        ],
        compiler_params=pltpu.CompilerParams(use_tc_tiling_on_sc=False),
    )
    def sc_agg(xa, xb, src2, dst2, za, zb, zs, sidx, didx, rows,
               gs0, gs1, isem, ss0, ss1):
        gsems = (gs0, gs1)
        ssems = (ss0, ss1)
        c = lax.axis_index("c")
        s = lax.axis_index("s")

        def run(xh, zh):
            pltpu.sync_copy(xh.at[pl.ds(s * rps, rps)],
                            zs.at[pl.ds(s * rps, rps)])
            plsc.subcore_barrier()

            def idx_fetch(i, slot):
                rbase = s * (nchunk * CH) + i * CH
                pltpu.async_copy(src2.at[pl.ds(rbase, CH)], sidx.at[slot],
                                 isem)
                pltpu.async_copy(dst2.at[pl.ds(rbase, CH)], didx.at[slot],
                                 isem)

            def idx_wait(slot):
                pltpu.make_async_copy(src2.at[pl.ds(0, CH)], sidx.at[slot],
                                      isem).wait()
                pltpu.make_async_copy(dst2.at[pl.ds(0, CH)], didx.at[slot],
                                      isem).wait()

            idx_fetch(0, 0)

            def chunk(i, carry):
                slot = i % 2
                idx_wait(slot)

                @pl.when(i + 1 < nchunk)
                def _():
                    idx_fetch(i + 1, (i + 1) % 2)

                si = sidx.at[slot]
                di = didx.at[slot]
                g = [None] * (CH + 2)
                sc = [None] * CH
                g[0] = pltpu.async_copy(xh.at[si.at[0]], rows.at[0],
                                        gsems[0])
                g[1] = pltpu.async_copy(xh.at[si.at[1]], rows.at[1],
                                        gsems[1])
                for j in range(CH):
                    g[j].wait()
                    sc[j] = pltpu.async_copy(
                        rows.at[j % 3], zs.at[di.at[j]], ssems[j % 2],
                        add=True)
                    if j >= 1:
                        sc[j - 1].wait()
                    if j + 2 < CH:
                        g[j + 2] = pltpu.async_copy(
                            xh.at[si.at[j + 2]], rows.at[(j + 2) % 3],
                            gsems[j % 2])
                sc[CH - 1].wait()
                return carry

            lax.fori_loop(0, nchunk, chunk, 0)
            plsc.subcore_barrier()
            pltpu.sync_copy(zs.at[pl.ds(s * rps, rps)],
                            zh.at[pl.ds(s * rps, rps)])

        @pl.when(c == 0)
        def _():
            run(xa, za)

        @pl.when(c == 1)
        def _():
            run(xb, zb)

    return sc_agg


def _bmm(a, b):
    return jnp.dot(a.astype(jnp.bfloat16), b.astype(jnp.bfloat16),
                   preferred_element_type=jnp.float32)


def _mlp1_body(za, zb, wa, ba, wb, bb, xa_o, xb_o):
    h = _bmm(za[...], wa[0:128, :]) + _bmm(zb[...], wa[128:256, :])
    h = jax.nn.relu(h + ba[...])
    o = _bmm(h, wb[...]) + bb[...]
    o = jax.nn.relu(o)
    xa_o[...] = o[:, :128]
    xb_o[...] = o[:, 128:]


def _mlp2_body(za, zb, wa, ba, wb, bb, x, x1a, x1b, out):
    h = _bmm(za[...], wa[0:128, :]) + _bmm(zb[...], wa[128:256, :])
    h = jax.nn.relu(h + ba[...])
    o = _bmm(h, wb[...]) + bb[...]
    x2 = jax.nn.relu(o)
    x1 = jnp.concatenate([x1a[...], x1b[...]], axis=1)
    out[...] = x[...] + x1 + x2


def kernel(x, edge_index, W1a, b1a, W1b, b1b, W2a, b2a, W2b, b2b):
    n, d = x.shape
    dh = d // 2
    h = W1a.shape[1]
    e = edge_index.shape[1]

    unit = NS * CH * B
    epad = ((e + unit - 1) // unit) * unit
    src = jnp.concatenate(
        [edge_index[0], jnp.zeros((epad - e,), jnp.int32)]).reshape(epad // B, B)
    dst = jnp.concatenate(
        [edge_index[1], jnp.full((epad - e,), n, jnp.int32)]).reshape(epad // B, B)

    xa = x[:, :dh]
    xb = x[:, dh:]

    sc_agg = _sc_agg_build(n, epad)

    bn = 1000
    grid = (n // bn,)
    row_spec = pl.BlockSpec((bn, dh), lambda i: (i, 0))
    wa_spec = pl.BlockSpec((d, h), lambda i: (0, 0))
    ba_spec = pl.BlockSpec((1, h), lambda i: (0, 0))
    wb_spec = pl.BlockSpec((h, d), lambda i: (0, 0))
    bb_spec = pl.BlockSpec((1, d), lambda i: (0, 0))

    mlp1 = pl.pallas_call(
        _mlp1_body,
        grid=grid,
        in_specs=[row_spec, row_spec, wa_spec, ba_spec, wb_spec, bb_spec],
        out_specs=[row_spec, row_spec],
        out_shape=[
            jax.ShapeDtypeStruct((n, dh), jnp.float32),
            jax.ShapeDtypeStruct((n, dh), jnp.float32),
        ],
    )
    mlp2 = pl.pallas_call(
        _mlp2_body,
        grid=grid,
        in_specs=[row_spec, row_spec, wa_spec, ba_spec, wb_spec, bb_spec,
                  pl.BlockSpec((bn, d), lambda i: (i, 0)), row_spec, row_spec],
        out_specs=pl.BlockSpec((bn, d), lambda i: (i, 0)),
        out_shape=jax.ShapeDtypeStruct((n, d), jnp.float32),
    )

    za, zb = sc_agg(xa, xb, src, dst)
    x1a, x1b = mlp1(za, zb, W1a, b1a.reshape(1, h), W1b, b1b.reshape(1, d))
    z2a, z2b = sc_agg(x1a, x1b, src, dst)
    out = mlp2(z2a, z2b, W2a, b2a.reshape(1, h), W2b, b2b.reshape(1, d),
               x, x1a, x1b)
    return out


# speedup vs baseline: 1.0052x; 1.0052x over previous
import functools

import jax
import jax.numpy as jnp
from jax import lax
from jax.experimental import pallas as pl
from jax.experimental.pallas import tpu as pltpu
from jax.experimental.pallas import tpu_sc as plsc

NS = 16
B = 112
CH = 15


def _sc_agg_build(n, epad):
    dh = 128
    nrow_pad = n + 8
    nchunk = epad // (NS * CH * B)
    rps = n // NS

    mesh = plsc.VectorSubcoreMesh(core_axis_name="c", subcore_axis_name="s",
                                  num_cores=2, num_subcores=NS)

    @functools.partial(
        pl.kernel,
        out_type=[
            jax.ShapeDtypeStruct((n, dh), jnp.float32),
            jax.ShapeDtypeStruct((n, dh), jnp.float32),
        ],
        mesh=mesh,
        scratch_types=[
            pltpu.VMEM_SHARED((nrow_pad, dh), jnp.float32),
            pltpu.VMEM((2, CH, B), jnp.int32),
            pltpu.VMEM((2, CH, B), jnp.int32),
            pltpu.VMEM((3, B, dh), jnp.float32),
            pltpu.SemaphoreType.DMA,
            pltpu.SemaphoreType.DMA,
            pltpu.SemaphoreType.DMA,
            pltpu.SemaphoreType.DMA,
            pltpu.SemaphoreType.DMA,
        ],
        compiler_params=pltpu.CompilerParams(use_tc_tiling_on_sc=False),
    )
    def sc_agg(xa, xb, src2, dst2, za, zb, zs, sidx, didx, rows,
               gs0, gs1, isem, ss0, ss1):
        gsems = (gs0, gs1)
        ssems = (ss0, ss1)
        c = lax.axis_index("c")
        s = lax.axis_index("s")

        def run(xh, zh):
            pltpu.sync_copy(xh.at[pl.ds(s * rps, rps)],
                            zs.at[pl.ds(s * rps, rps)])
            plsc.subcore_barrier()

            def idx_fetch(i, slot):
                rbase = s * (nchunk * CH) + i * CH
                pltpu.async_copy(src2.at[pl.ds(rbase, CH)], sidx.at[slot],
                                 isem)
                pltpu.async_copy(dst2.at[pl.ds(rbase, CH)], didx.at[slot],
                                 isem)

            def idx_wait(slot):
                pltpu.make_async_copy(src2.at[pl.ds(0, CH)], sidx.at[slot],
                                      isem).wait()
                pltpu.make_async_copy(dst2.at[pl.ds(0, CH)], didx.at[slot],
                                      isem).wait()

            idx_fetch(0, 0)

            def chunk(i, carry):
                slot = i % 2
                idx_wait(slot)

                @pl.when(i + 1 < nchunk)
                def _():
                    idx_fetch(i + 1, (i + 1) % 2)

                si = sidx.at[slot]
                di = didx.at[slot]
                g = [None] * (CH + 2)
                sc = [None] * CH
                g[0] = pltpu.async_copy(xh.at[si.at[0]], rows.at[0],
                                        gsems[0])
                g[1] = pltpu.async_copy(xh.at[si.at[1]], rows.at[1],
                                        gsems[1])
                for j in range(CH):
                    g[j].wait()
                    sc[j] = pltpu.async_copy(
                        rows.at[j % 3], zs.at[di.at[j]], ssems[j % 2],
                        add=True)
                    if j >= 1:
                        sc[j - 1].wait()
                    if j + 2 < CH:
                        g[j + 2] = pltpu.async_copy(
                            xh.at[si.at[j + 2]], rows.at[(j + 2) % 3],
                            gsems[j % 2])
                sc[CH - 1].wait()
                return carry

            lax.fori_loop(0, nchunk, chunk, 0)
            plsc.subcore_barrier()
            pltpu.sync_copy(zs.at[pl.ds(s * rps, rps)],
                            zh.at[pl.ds(s * rps, rps)])

        @pl.when(c == 0)
        def _():
            run(xa, za)

        @pl.when(c == 1)
        def _():
            run(xb, zb)

    return sc_agg


def _mlp1_body(za, zb, wa, ba, wb, bb, xa_o, xb_o):
    h = jnp.dot(za[...], wa[0:128, :], preferred_element_type=jnp.float32)
    h = h + jnp.dot(zb[...], wa[128:256, :], preferred_element_type=jnp.float32)
    h = jax.nn.relu(h + ba[...])
    o = jnp.dot(h, wb[...], preferred_element_type=jnp.float32) + bb[...]
    o = jax.nn.relu(o)
    xa_o[...] = o[:, :128]
    xb_o[...] = o[:, 128:]


def _mlp2_body(za, zb, wa, ba, wb, bb, x, x1a, x1b, out):
    h = jnp.dot(za[...], wa[0:128, :], preferred_element_type=jnp.float32)
    h = h + jnp.dot(zb[...], wa[128:256, :], preferred_element_type=jnp.float32)
    h = jax.nn.relu(h + ba[...])
    o = jnp.dot(h, wb[...], preferred_element_type=jnp.float32) + bb[...]
    x2 = jax.nn.relu(o)
    x1 = jnp.concatenate([x1a[...], x1b[...]], axis=1)
    out[...] = x[...] + x1 + x2


def kernel(x, edge_index, W1a, b1a, W1b, b1b, W2a, b2a, W2b, b2b):
    n, d = x.shape
    dh = d // 2
    h = W1a.shape[1]
    e = edge_index.shape[1]

    unit = NS * CH * B
    epad = ((e + unit - 1) // unit) * unit
    src = jnp.concatenate(
        [edge_index[0], jnp.zeros((epad - e,), jnp.int32)]).reshape(epad // B, B)
    dst = jnp.concatenate(
        [edge_index[1], jnp.full((epad - e,), n, jnp.int32)]).reshape(epad // B, B)

    xa = x[:, :dh]
    xb = x[:, dh:]

    sc_agg = _sc_agg_build(n, epad)

    bn = 1000
    grid = (n // bn,)
    row_spec = pl.BlockSpec((bn, dh), lambda i: (i, 0))
    wa_spec = pl.BlockSpec((d, h), lambda i: (0, 0))
    ba_spec = pl.BlockSpec((1, h), lambda i: (0, 0))
    wb_spec = pl.BlockSpec((h, d), lambda i: (0, 0))
    bb_spec = pl.BlockSpec((1, d), lambda i: (0, 0))

    mlp1 = pl.pallas_call(
        _mlp1_body,
        grid=grid,
        in_specs=[row_spec, row_spec, wa_spec, ba_spec, wb_spec, bb_spec],
        out_specs=[row_spec, row_spec],
        out_shape=[
            jax.ShapeDtypeStruct((n, dh), jnp.float32),
            jax.ShapeDtypeStruct((n, dh), jnp.float32),
        ],
    )
    mlp2 = pl.pallas_call(
        _mlp2_body,
        grid=grid,
        in_specs=[row_spec, row_spec, wa_spec, ba_spec, wb_spec, bb_spec,
                  pl.BlockSpec((bn, d), lambda i: (i, 0)), row_spec, row_spec],
        out_specs=pl.BlockSpec((bn, d), lambda i: (i, 0)),
        out_shape=jax.ShapeDtypeStruct((n, d), jnp.float32),
    )

    za, zb = sc_agg(xa, xb, src, dst)
    x1a, x1b = mlp1(za, zb, W1a, b1a.reshape(1, h), W1b, b1b.reshape(1, d))
    z2a, z2b = sc_agg(x1a, x1b, src, dst)
    out = mlp2(z2a, z2b, W2a, b2a.reshape(1, h), W2b, b2b.reshape(1, d),
               x, x1a, x1b)
    return out
